# baseline (speedup 1.0000x reference)
import functools

import jax
import jax.numpy as jnp
from jax import lax
from jax.experimental import pallas as pl
from jax.experimental.pallas import tpu as pltpu
from jax.experimental.pallas import tpu_sc as plsc

NUM_USERS = 2500
NUM_ITEMS = 7500
N_NODES = NUM_USERS + NUM_ITEMS
D = 128
E_A = 320000
E_S = 80000

CHUNK = 80
NC, NS = 2, 16
NW = NC * NS
U_PAD = 2560
N_PAD = 10240
A_ROWS_PER_TILE = N_PAD // NS
S_ROWS_PER_TILE = U_PAD // NS
T_A = 128
T_S = 32
E_A_PAD = NW * T_A * CHUNK
E_S_PAD = NW * T_S * CHUNK
DEPTH = 4


def _spmm_pair_body(a_src, a_dst, a_val, s_src, s_dst, s_val,
                    x_ego, x_soc, zeros_hbm,
                    out_a, out_s,
                    acc, srcb, dstb, valb,
                    rows0, rows1, rows2, rows3,
                    gsem0, gsem1, gsem2, gsem3,
                    ssem0, ssem1, ssem2, ssem3,
                    isem0, isem1, isem2, isem3):
    cid = lax.axis_index("c")
    sid = lax.axis_index("s")
    wid = sid * NC + cid
    rows = (rows0, rows1, rows2, rows3)
    gsem = (gsem0, gsem1, gsem2, gsem3)
    ssem = (ssem0, ssem1, ssem2, ssem3)
    isem = (isem0, isem1, isem2, isem3)

    def zero_acc(nrows_per_tile):
        pltpu.sync_copy(
            zeros_hbm.at[pl.ds(sid * nrows_per_tile, nrows_per_tile)],
            acc.at[pl.ds(sid * nrows_per_tile, nrows_per_tile)])

    def run_set(pack_s, pack_d, pack_v, x_hbm, T, ebase):
        def stage(k, b):
            off = (ebase + k) * CHUNK
            pltpu.async_copy(pack_s.at[pl.ds(off, CHUNK)], srcb.at[b], isem[b])
            pltpu.async_copy(pack_d.at[pl.ds(off, CHUNK)], dstb.at[b], isem[b])
            pltpu.async_copy(pack_v.at[pl.ds(off, CHUNK)], valb.at[b], isem[b])

        def stage_wait(b):
            for ref in (srcb, dstb, valb):
                pltpu.make_async_copy(pack_s.at[pl.ds(0, CHUNK)], ref.at[b],
                                      isem[b]).wait()

        def gather(b):
            pltpu.async_copy(x_hbm.at[srcb.at[b]], rows[b], gsem[b])

        def gwait(b):
            pltpu.make_async_copy(x_hbm.at[pl.ds(0, CHUNK)], rows[b],
                                  gsem[b]).wait()

        def scatter(b):
            pltpu.async_copy(rows[b], acc.at[dstb.at[b]], ssem[b], add=True)

        def swait(b):
            pltpu.make_async_copy(x_hbm.at[pl.ds(0, CHUNK)], rows[b],
                                  ssem[b]).wait()

        def scale(b):
            r = rows[b]

            def group_scale(g, c2):
                vv = valb[b, pl.ds(g * 16, 16)]
                for i in range(16):
                    bc = vv.at[jnp.full((16,), i, jnp.int32)].get(
                        mode="promise_in_bounds")
                    rr = g * 16 + i
                    for h in range(D // 16):
                        sl = pl.ds(h * 16, 16)
                        r[rr, sl] = r[rr, sl] * bc
                return c2
            lax.fori_loop(0, CHUNK // 16, group_scale, 0)

        for b in (0, 1):
            stage(b, b)
            stage_wait(b)
            gather(b)

        def ring(i, carry):
            k0 = i * DEPTH
            for bb in range(DEPTH):
                k = k0 + bb
                nb = (bb + 2) % DEPTH

                @pl.when(k >= 2)
                def _():
                    swait(nb)

                @pl.when(k + 2 < T)
                def _():
                    stage(k + 2, nb)
                    stage_wait(nb)
                    gather(nb)
                gwait(bb)
                scale(bb)
                scatter(bb)
            return carry
        lax.fori_loop(0, T // DEPTH, ring, 0)
        swait((T - 2) % DEPTH)
        swait((T - 1) % DEPTH)

    zero_acc(A_ROWS_PER_TILE)
    plsc.subcore_barrier()
    run_set(a_src, a_dst, a_val, x_ego, T_A, wid * T_A)
    plsc.subcore_barrier()
    pltpu.sync_copy(acc.at[pl.ds(sid * A_ROWS_PER_TILE, A_ROWS_PER_TILE)],
                    out_a.at[cid, pl.ds(sid * A_ROWS_PER_TILE, A_ROWS_PER_TILE)])
    plsc.subcore_barrier()

    zero_acc(S_ROWS_PER_TILE)
    plsc.subcore_barrier()
    run_set(s_src, s_dst, s_val, x_soc, T_S, wid * T_S)
    plsc.subcore_barrier()
    pltpu.sync_copy(acc.at[pl.ds(sid * S_ROWS_PER_TILE, S_ROWS_PER_TILE)],
                    out_s.at[cid, pl.ds(sid * S_ROWS_PER_TILE, S_ROWS_PER_TILE)])


_spmm_pair = pl.kernel(
    _spmm_pair_body,
    out_type=(jax.ShapeDtypeStruct((NC, N_PAD, D), jnp.float32),
              jax.ShapeDtypeStruct((NC, U_PAD, D), jnp.float32)),
    mesh=plsc.VectorSubcoreMesh(core_axis_name="c", subcore_axis_name="s"),
    scratch_types=[
        pltpu.VMEM_SHARED((N_PAD, D), jnp.float32),
        pltpu.VMEM((DEPTH, CHUNK), jnp.int32),
        pltpu.VMEM((DEPTH, CHUNK), jnp.int32),
        pltpu.VMEM((DEPTH, CHUNK), jnp.float32),
        pltpu.VMEM((CHUNK, D), jnp.float32),
        pltpu.VMEM((CHUNK, D), jnp.float32),
        pltpu.VMEM((CHUNK, D), jnp.float32),
        pltpu.VMEM((CHUNK, D), jnp.float32),
        pltpu.SemaphoreType.DMA,
        pltpu.SemaphoreType.DMA,
        pltpu.SemaphoreType.DMA,
        pltpu.SemaphoreType.DMA,
        pltpu.SemaphoreType.DMA,
        pltpu.SemaphoreType.DMA,
        pltpu.SemaphoreType.DMA,
        pltpu.SemaphoreType.DMA,
        pltpu.SemaphoreType.DMA,
        pltpu.SemaphoreType.DMA,
        pltpu.SemaphoreType.DMA,
        pltpu.SemaphoreType.DMA,
    ],
)


def _mish(x):
    return x * jnp.tanh(jax.nn.softplus(x))


def _fusion_core(ep_ref, sp_ref, w1, b1, w2, b2, w3, b3):
    u = ep_ref[0, :NUM_USERS, :] + ep_ref[1, :NUM_USERS, :]
    s = sp_ref[0, :NUM_USERS, :] + sp_ref[1, :NUM_USERS, :]
    c = jnp.concatenate([u, s, u * s], axis=1)
    t1 = _mish(jnp.dot(c, w1[...], preferred_element_type=jnp.float32) + b1[...])
    t2 = _mish(jnp.dot(t1, w2[...], preferred_element_type=jnp.float32) + b2[...])
    t3 = jnp.dot(t2, w3[...], preferred_element_type=jnp.float32) + b3[...]
    soc = t3 / jnp.sqrt(jnp.sum(t3 * t3))
    items = ep_ref[0, NUM_USERS:N_NODES, :] + ep_ref[1, NUM_USERS:N_NODES, :]
    return soc, items


def _fusion_mid_body(ep_ref, sp_ref, w1, b1, w2, b2, w3, b3, ego_out, soc_out):
    soc, items = _fusion_core(ep_ref, sp_ref, w1, b1, w2, b2, w3, b3)
    ego_out[:NUM_USERS, :] = soc
    ego_out[NUM_USERS:, :] = items
    soc_out[...] = soc


def _fusion_final_body(ep_ref, sp_ref, w1, b1, w2, b2, w3, b3,
                       ego0_ref, ego1_ref, u_out, i_out):
    soc, items = _fusion_core(ep_ref, sp_ref, w1, b1, w2, b2, w3, b3)
    u_out[...] = (ego0_ref[:NUM_USERS, :] + ego1_ref[:NUM_USERS, :] + soc) / 3.0
    i_out[...] = (ego0_ref[NUM_USERS:, :] + ego1_ref[NUM_USERS:, :] + items) / 3.0


_fusion_mid = pl.pallas_call(
    _fusion_mid_body,
    out_shape=(jax.ShapeDtypeStruct((N_NODES, D), jnp.float32),
               jax.ShapeDtypeStruct((NUM_USERS, D), jnp.float32)),
)

_fusion_final = pl.pallas_call(
    _fusion_final_body,
    out_shape=(jax.ShapeDtypeStruct((NUM_USERS, D), jnp.float32),
               jax.ShapeDtypeStruct((NUM_ITEMS, D), jnp.float32)),
)


def _pack(x, e_pad):
    return jnp.pad(x, (0, e_pad - x.shape[0]))


def kernel(user_emb, item_emb, a_vals, s_vals, fc1_w, fc1_b, fc2_w, fc2_b,
           fc3_w, fc3_b, edge_index_a, edge_index_s):
    ego0 = jnp.concatenate([user_emb, item_emb], axis=0)
    soc0 = user_emb
    zeros = jnp.zeros((N_PAD, D), jnp.float32)
    a_src = _pack(edge_index_a[0], E_A_PAD)
    a_dst = _pack(edge_index_a[1], E_A_PAD)
    a_val = _pack(a_vals, E_A_PAD)
    s_src = _pack(edge_index_s[0], E_S_PAD)
    s_dst = _pack(edge_index_s[1], E_S_PAD)
    s_val = _pack(s_vals, E_S_PAD)

    ep1, sp1 = _spmm_pair(a_src, a_dst, a_val, s_src, s_dst, s_val,
                          ego0, soc0, zeros)
    ego1, soc1 = _fusion_mid(ep1, sp1, fc1_w, fc1_b, fc2_w, fc2_b, fc3_w, fc3_b)
    ep2, sp2 = _spmm_pair(a_src, a_dst, a_val, s_src, s_dst, s_val,
                          ego1, soc1, zeros)
    u_mean, i_mean = _fusion_final(ep2, sp2, fc1_w, fc1_b, fc2_w, fc2_b,
                                   fc3_w, fc3_b, ego0, ego1)
    return u_mean, i_mean


# speedup vs baseline: 2.7290x; 2.7290x over previous
import functools

import jax
import jax.numpy as jnp
from jax import lax
from jax.experimental import pallas as pl
from jax.experimental.pallas import tpu as pltpu
from jax.experimental.pallas import tpu_sc as plsc

NUM_USERS = 2500
NUM_ITEMS = 7500
N_NODES = NUM_USERS + NUM_ITEMS
D = 128
E_A = 320000
E_S = 80000

CHUNK = 80
NC, NS = 2, 16
NW = NC * NS
U_PAD = 2560
N_PAD = 10240
A_ROWS_PER_TILE = N_PAD // NS
S_ROWS_PER_TILE = U_PAD // NS
T_A = 128
T_S = 32
E_A_PAD = NW * T_A * CHUNK
E_S_PAD = NW * T_S * CHUNK
DEPTH = 4


def _spmm_pair_body(a_src, a_dst, a_val, s_src, s_dst, s_val,
                    x_ego, x_soc, zeros_hbm,
                    out_a, out_s,
                    acc, srcb, dstb, valb,
                    rows0, rows1, rows2, rows3,
                    gsem0, gsem1, gsem2, gsem3,
                    ssem0, ssem1, ssem2, ssem3,
                    isem0, isem1, isem2, isem3):
    cid = lax.axis_index("c")
    sid = lax.axis_index("s")
    wid = sid * NC + cid
    rows = (rows0, rows1, rows2, rows3)
    gsem = (gsem0, gsem1, gsem2, gsem3)
    ssem = (ssem0, ssem1, ssem2, ssem3)
    isem = (isem0, isem1, isem2, isem3)

    def zero_acc(nrows_per_tile):
        pltpu.sync_copy(
            zeros_hbm.at[pl.ds(sid * nrows_per_tile, nrows_per_tile)],
            acc.at[pl.ds(sid * nrows_per_tile, nrows_per_tile)])

    def run_set(pack_s, pack_d, pack_v, x_hbm, T, ebase):
        def stage(k, b):
            off = (ebase + k) * CHUNK
            pltpu.async_copy(pack_s.at[pl.ds(off, CHUNK)], srcb.at[b], isem[b])
            pltpu.async_copy(pack_d.at[pl.ds(off, CHUNK)], dstb.at[b], isem[b])
            pltpu.async_copy(pack_v.at[pl.ds(off, CHUNK)], valb.at[b], isem[b])

        def stage_wait(b):
            for ref in (srcb, dstb, valb):
                pltpu.make_async_copy(pack_s.at[pl.ds(0, CHUNK)], ref.at[b],
                                      isem[b]).wait()

        def gather(b):
            pltpu.async_copy(x_hbm.at[srcb.at[b]], rows[b], gsem[b])

        def gwait(b):
            pltpu.make_async_copy(x_hbm.at[pl.ds(0, CHUNK)], rows[b],
                                  gsem[b]).wait()

        def scatter(b):
            pltpu.async_copy(rows[b], acc.at[dstb.at[b]], ssem[b], add=True)

        def swait(b):
            pltpu.make_async_copy(x_hbm.at[pl.ds(0, CHUNK)], rows[b],
                                  ssem[b]).wait()

        def scale(b):
            r = rows[b]

            def group_scale(g, c2):
                vv = valb[b, pl.ds(g * 16, 16)]
                for i in range(16):
                    bc = vv.at[jnp.full((16,), i, jnp.int32)].get(
                        mode="promise_in_bounds")
                    rr = g * 16 + i
                    for h in range(D // 16):
                        sl = pl.ds(h * 16, 16)
                        r[rr, sl] = r[rr, sl] * bc
                return c2
            lax.fori_loop(0, CHUNK // 16, group_scale, 0)

        for b in (0, 1):
            stage(b, b)
            stage_wait(b)
            gather(b)

        def ring(i, carry):
            k0 = i * DEPTH
            for bb in range(DEPTH):
                k = k0 + bb
                nb = (bb + 2) % DEPTH

                @pl.when(k >= 2)
                def _():
                    swait(nb)

                @pl.when(k + 2 < T)
                def _():
                    stage(k + 2, nb)
                    stage_wait(nb)
                    gather(nb)
                gwait(bb)
                scale(bb)
                scatter(bb)
            return carry
        lax.fori_loop(0, T // DEPTH, ring, 0)
        swait((T - 2) % DEPTH)
        swait((T - 1) % DEPTH)

    zero_acc(A_ROWS_PER_TILE)
    plsc.subcore_barrier()
    run_set(a_src, a_dst, a_val, x_ego, T_A, wid * T_A)
    plsc.subcore_barrier()
    pltpu.sync_copy(acc.at[pl.ds(sid * A_ROWS_PER_TILE, A_ROWS_PER_TILE)],
                    out_a.at[cid, pl.ds(sid * A_ROWS_PER_TILE, A_ROWS_PER_TILE)])
    plsc.subcore_barrier()

    zero_acc(S_ROWS_PER_TILE)
    plsc.subcore_barrier()
    run_set(s_src, s_dst, s_val, x_soc, T_S, wid * T_S)
    plsc.subcore_barrier()
    pltpu.sync_copy(acc.at[pl.ds(sid * S_ROWS_PER_TILE, S_ROWS_PER_TILE)],
                    out_s.at[cid, pl.ds(sid * S_ROWS_PER_TILE, S_ROWS_PER_TILE)])


_spmm_pair = pl.kernel(
    _spmm_pair_body,
    out_type=(jax.ShapeDtypeStruct((NC, N_PAD, D), jnp.float32),
              jax.ShapeDtypeStruct((NC, U_PAD, D), jnp.float32)),
    mesh=plsc.VectorSubcoreMesh(core_axis_name="c", subcore_axis_name="s"),
    scratch_types=[
        pltpu.VMEM_SHARED((N_PAD, D), jnp.float32),
        pltpu.VMEM((DEPTH, CHUNK), jnp.int32),
        pltpu.VMEM((DEPTH, CHUNK), jnp.int32),
        pltpu.VMEM((DEPTH, CHUNK), jnp.float32),
        pltpu.VMEM((CHUNK, D), jnp.float32),
        pltpu.VMEM((CHUNK, D), jnp.float32),
        pltpu.VMEM((CHUNK, D), jnp.float32),
        pltpu.VMEM((CHUNK, D), jnp.float32),
        pltpu.SemaphoreType.DMA,
        pltpu.SemaphoreType.DMA,
        pltpu.SemaphoreType.DMA,
        pltpu.SemaphoreType.DMA,
        pltpu.SemaphoreType.DMA,
        pltpu.SemaphoreType.DMA,
        pltpu.SemaphoreType.DMA,
        pltpu.SemaphoreType.DMA,
        pltpu.SemaphoreType.DMA,
        pltpu.SemaphoreType.DMA,
        pltpu.SemaphoreType.DMA,
        pltpu.SemaphoreType.DMA,
    ],
)


def _mish(x):
    return x * jnp.tanh(jax.nn.softplus(x))


def _fusion_core(ep_ref, sp_ref, w1, b1, w2, b2, w3, b3):
    u = ep_ref[0, :NUM_USERS, :] + ep_ref[1, :NUM_USERS, :]
    s = sp_ref[0, :NUM_USERS, :] + sp_ref[1, :NUM_USERS, :]
    c = jnp.concatenate([u, s, u * s], axis=1)
    t1 = _mish(jnp.dot(c, w1[...], preferred_element_type=jnp.float32) + b1[...])
    t2 = _mish(jnp.dot(t1, w2[...], preferred_element_type=jnp.float32) + b2[...])
    t3 = jnp.dot(t2, w3[...], preferred_element_type=jnp.float32) + b3[...]
    soc = t3 / jnp.sqrt(jnp.sum(t3 * t3))
    items = ep_ref[0, NUM_USERS:N_NODES, :] + ep_ref[1, NUM_USERS:N_NODES, :]
    return soc, items


def _fusion_mid_body(ep_ref, sp_ref, w1, b1, w2, b2, w3, b3, ego_out, soc_out):
    soc, items = _fusion_core(ep_ref, sp_ref, w1, b1, w2, b2, w3, b3)
    ego_out[:NUM_USERS, :] = soc
    ego_out[NUM_USERS:, :] = items
    soc_out[...] = soc


def _fusion_final_body(ep_ref, sp_ref, w1, b1, w2, b2, w3, b3,
                       ego0_ref, ego1_ref, u_out, i_out):
    soc, items = _fusion_core(ep_ref, sp_ref, w1, b1, w2, b2, w3, b3)
    u_out[...] = (ego0_ref[:NUM_USERS, :] + ego1_ref[:NUM_USERS, :] + soc) / 3.0
    i_out[...] = (ego0_ref[NUM_USERS:, :] + ego1_ref[NUM_USERS:, :] + items) / 3.0


_fusion_mid = pl.pallas_call(
    _fusion_mid_body,
    out_shape=(jax.ShapeDtypeStruct((N_NODES, D), jnp.float32),
               jax.ShapeDtypeStruct((NUM_USERS, D), jnp.float32)),
)

_fusion_final = pl.pallas_call(
    _fusion_final_body,
    out_shape=(jax.ShapeDtypeStruct((NUM_USERS, D), jnp.float32),
               jax.ShapeDtypeStruct((NUM_ITEMS, D), jnp.float32)),
)


def _pack(x, e_pad):
    return jnp.pad(x, (0, e_pad - x.shape[0]))


def _pack_idx(x, e_pad, spread_base, spread_n):
    npad = e_pad - x.shape[0]
    fill = spread_base + (jnp.arange(npad, dtype=jnp.int32) % spread_n)
    return jnp.concatenate([x, fill])


def kernel(user_emb, item_emb, a_vals, s_vals, fc1_w, fc1_b, fc2_w, fc2_b,
           fc3_w, fc3_b, edge_index_a, edge_index_s):
    ego0 = jnp.concatenate([user_emb, item_emb], axis=0)
    soc0 = user_emb
    zeros = jnp.zeros((N_PAD, D), jnp.float32)
    a_src = _pack_idx(edge_index_a[0], E_A_PAD, 0, N_NODES)
    a_dst = _pack_idx(edge_index_a[1], E_A_PAD, N_NODES, N_PAD - N_NODES)
    a_val = _pack(a_vals, E_A_PAD)
    s_src = _pack_idx(edge_index_s[0], E_S_PAD, 0, NUM_USERS)
    s_dst = _pack_idx(edge_index_s[1], E_S_PAD, NUM_USERS, U_PAD - NUM_USERS)
    s_val = _pack(s_vals, E_S_PAD)

    ep1, sp1 = _spmm_pair(a_src, a_dst, a_val, s_src, s_dst, s_val,
                          ego0, soc0, zeros)
    ego1, soc1 = _fusion_mid(ep1, sp1, fc1_w, fc1_b, fc2_w, fc2_b, fc3_w, fc3_b)
    ep2, sp2 = _spmm_pair(a_src, a_dst, a_val, s_src, s_dst, s_val,
                          ego1, soc1, zeros)
    u_mean, i_mean = _fusion_final(ep2, sp2, fc1_w, fc1_b, fc2_w, fc2_b,
                                   fc3_w, fc3_b, ego0, ego1)
    return u_mean, i_mean


# speedup vs baseline: 3.0146x; 1.1047x over previous
import functools

import jax
import jax.numpy as jnp
from jax import lax
from jax.experimental import pallas as pl
from jax.experimental.pallas import tpu as pltpu
from jax.experimental.pallas import tpu_sc as plsc

NUM_USERS = 2500
NUM_ITEMS = 7500
N_NODES = NUM_USERS + NUM_ITEMS
D = 128
E_A = 320000
E_S = 80000

CHUNK = 80
NC, NS = 2, 16
NW = NC * NS
U_PAD = 2560
N_PAD = 10240
A_ROWS_PER_TILE = N_PAD // NS
S_ROWS_PER_TILE = U_PAD // NS
T_A = 128
T_S = 32
E_A_PAD = NW * T_A * CHUNK
E_S_PAD = NW * T_S * CHUNK
DEPTH = 4


def _spmm_pair_body(a_src, a_dst, a_val, s_src, s_dst, s_val,
                    x_ego, x_soc, zeros_hbm,
                    out_a, out_s,
                    acc, srcb, dstb, valb,
                    rows0, rows1, rows2, rows3,
                    gsem0, gsem1, gsem2, gsem3,
                    ssem0, ssem1, ssem2, ssem3,
                    isem0, isem1, isem2, isem3):
    cid = lax.axis_index("c")
    sid = lax.axis_index("s")
    wid = sid * NC + cid
    rows = (rows0, rows1, rows2, rows3)
    gsem = (gsem0, gsem1, gsem2, gsem3)
    ssem = (ssem0, ssem1, ssem2, ssem3)
    isem = (isem0, isem1, isem2, isem3)

    def zero_acc(nrows_per_tile):
        pltpu.sync_copy(
            zeros_hbm.at[pl.ds(sid * nrows_per_tile, nrows_per_tile)],
            acc.at[pl.ds(sid * nrows_per_tile, nrows_per_tile)])

    def run_set(pack_s, pack_d, pack_v, x_hbm, T, ebase):
        def stage(k, b):
            off = (ebase + k) * CHUNK
            pltpu.async_copy(pack_s.at[pl.ds(off, CHUNK)], srcb.at[b], isem[b])
            pltpu.async_copy(pack_d.at[pl.ds(off, CHUNK)], dstb.at[b], isem[b])
            pltpu.async_copy(pack_v.at[pl.ds(off, CHUNK)], valb.at[b], isem[b])

        def stage_wait(b):
            for ref in (srcb, dstb, valb):
                pltpu.make_async_copy(pack_s.at[pl.ds(0, CHUNK)], ref.at[b],
                                      isem[b]).wait()

        def gather(b):
            pltpu.async_copy(x_hbm.at[srcb.at[b]], rows[b], gsem[b])

        def gwait(b):
            pltpu.make_async_copy(x_hbm.at[pl.ds(0, CHUNK)], rows[b],
                                  gsem[b]).wait()

        def scatter(b):
            pltpu.async_copy(rows[b], acc.at[dstb.at[b]], ssem[b], add=True)

        def swait(b):
            pltpu.make_async_copy(x_hbm.at[pl.ds(0, CHUNK)], rows[b],
                                  ssem[b]).wait()

        def scale(b):
            r = rows[b]

            def group_scale(g, c2):
                vv = valb[b, pl.ds(g * 16, 16)]
                for i in range(16):
                    bc = vv.at[jnp.full((16,), i, jnp.int32)].get(
                        mode="promise_in_bounds")
                    rr = g * 16 + i
                    for h in range(D // 16):
                        sl = pl.ds(h * 16, 16)
                        r[rr, sl] = r[rr, sl] * bc
                return c2
            lax.fori_loop(0, CHUNK // 16, group_scale, 0)

        for b in (0, 1):
            stage(b, b)
            stage_wait(b)
            gather(b)

        def ring(i, carry):
            k0 = i * DEPTH
            for bb in range(DEPTH):
                k = k0 + bb
                nb = (bb + 2) % DEPTH

                @pl.when(k >= 2)
                def _():
                    swait(nb)

                @pl.when(k + 2 < T)
                def _():
                    stage(k + 2, nb)
                gwait(bb)
                scale(bb)

                @pl.when(k + 2 < T)
                def _():
                    stage_wait(nb)
                    gather(nb)
                scatter(bb)
            return carry
        lax.fori_loop(0, T // DEPTH, ring, 0)
        swait((T - 2) % DEPTH)
        swait((T - 1) % DEPTH)

    zero_acc(A_ROWS_PER_TILE)
    plsc.subcore_barrier()
    run_set(a_src, a_dst, a_val, x_ego, T_A, wid * T_A)
    plsc.subcore_barrier()
    pltpu.sync_copy(acc.at[pl.ds(sid * A_ROWS_PER_TILE, A_ROWS_PER_TILE)],
                    out_a.at[cid, pl.ds(sid * A_ROWS_PER_TILE, A_ROWS_PER_TILE)])
    plsc.subcore_barrier()

    zero_acc(S_ROWS_PER_TILE)
    plsc.subcore_barrier()
    run_set(s_src, s_dst, s_val, x_soc, T_S, wid * T_S)
    plsc.subcore_barrier()
    pltpu.sync_copy(acc.at[pl.ds(sid * S_ROWS_PER_TILE, S_ROWS_PER_TILE)],
                    out_s.at[cid, pl.ds(sid * S_ROWS_PER_TILE, S_ROWS_PER_TILE)])


_spmm_pair = pl.kernel(
    _spmm_pair_body,
    out_type=(jax.ShapeDtypeStruct((NC, N_PAD, D), jnp.float32),
              jax.ShapeDtypeStruct((NC, U_PAD, D), jnp.float32)),
    mesh=plsc.VectorSubcoreMesh(core_axis_name="c", subcore_axis_name="s"),
    scratch_types=[
        pltpu.VMEM_SHARED((N_PAD, D), jnp.float32),
        pltpu.VMEM((DEPTH, CHUNK), jnp.int32),
        pltpu.VMEM((DEPTH, CHUNK), jnp.int32),
        pltpu.VMEM((DEPTH, CHUNK), jnp.float32),
        pltpu.VMEM((CHUNK, D), jnp.float32),
        pltpu.VMEM((CHUNK, D), jnp.float32),
        pltpu.VMEM((CHUNK, D), jnp.float32),
        pltpu.VMEM((CHUNK, D), jnp.float32),
        pltpu.SemaphoreType.DMA,
        pltpu.SemaphoreType.DMA,
        pltpu.SemaphoreType.DMA,
        pltpu.SemaphoreType.DMA,
        pltpu.SemaphoreType.DMA,
        pltpu.SemaphoreType.DMA,
        pltpu.SemaphoreType.DMA,
        pltpu.SemaphoreType.DMA,
        pltpu.SemaphoreType.DMA,
        pltpu.SemaphoreType.DMA,
        pltpu.SemaphoreType.DMA,
        pltpu.SemaphoreType.DMA,
    ],
)


def _mish(x):
    return x * jnp.tanh(jax.nn.softplus(x))


def _fusion_core(ep_ref, sp_ref, w1, b1, w2, b2, w3, b3):
    u = ep_ref[0, :NUM_USERS, :] + ep_ref[1, :NUM_USERS, :]
    s = sp_ref[0, :NUM_USERS, :] + sp_ref[1, :NUM_USERS, :]
    c = jnp.concatenate([u, s, u * s], axis=1)
    t1 = _mish(jnp.dot(c, w1[...], preferred_element_type=jnp.float32) + b1[...])
    t2 = _mish(jnp.dot(t1, w2[...], preferred_element_type=jnp.float32) + b2[...])
    t3 = jnp.dot(t2, w3[...], preferred_element_type=jnp.float32) + b3[...]
    soc = t3 / jnp.sqrt(jnp.sum(t3 * t3))
    items = ep_ref[0, NUM_USERS:N_NODES, :] + ep_ref[1, NUM_USERS:N_NODES, :]
    return soc, items


def _fusion_mid_body(ep_ref, sp_ref, w1, b1, w2, b2, w3, b3, ego_out, soc_out):
    soc, items = _fusion_core(ep_ref, sp_ref, w1, b1, w2, b2, w3, b3)
    ego_out[:NUM_USERS, :] = soc
    ego_out[NUM_USERS:, :] = items
    soc_out[...] = soc


def _fusion_final_body(ep_ref, sp_ref, w1, b1, w2, b2, w3, b3,
                       ego0_ref, ego1_ref, u_out, i_out):
    soc, items = _fusion_core(ep_ref, sp_ref, w1, b1, w2, b2, w3, b3)
    u_out[...] = (ego0_ref[:NUM_USERS, :] + ego1_ref[:NUM_USERS, :] + soc) / 3.0
    i_out[...] = (ego0_ref[NUM_USERS:, :] + ego1_ref[NUM_USERS:, :] + items) / 3.0


_fusion_mid = pl.pallas_call(
    _fusion_mid_body,
    out_shape=(jax.ShapeDtypeStruct((N_NODES, D), jnp.float32),
               jax.ShapeDtypeStruct((NUM_USERS, D), jnp.float32)),
)

_fusion_final = pl.pallas_call(
    _fusion_final_body,
    out_shape=(jax.ShapeDtypeStruct((NUM_USERS, D), jnp.float32),
               jax.ShapeDtypeStruct((NUM_ITEMS, D), jnp.float32)),
)


def _pack(x, e_pad):
    return jnp.pad(x, (0, e_pad - x.shape[0]))


def _pack_idx(x, e_pad, spread_base, spread_n):
    npad = e_pad - x.shape[0]
    fill = spread_base + (jnp.arange(npad, dtype=jnp.int32) % spread_n)
    return jnp.concatenate([x, fill])


def kernel(user_emb, item_emb, a_vals, s_vals, fc1_w, fc1_b, fc2_w, fc2_b,
           fc3_w, fc3_b, edge_index_a, edge_index_s):
    ego0 = jnp.concatenate([user_emb, item_emb], axis=0)
    soc0 = user_emb
    zeros = jnp.zeros((N_PAD, D), jnp.float32)
    a_src = _pack_idx(edge_index_a[0], E_A_PAD, 0, N_NODES)
    a_dst = _pack_idx(edge_index_a[1], E_A_PAD, N_NODES, N_PAD - N_NODES)
    a_val = _pack(a_vals, E_A_PAD)
    s_src = _pack_idx(edge_index_s[0], E_S_PAD, 0, NUM_USERS)
    s_dst = _pack_idx(edge_index_s[1], E_S_PAD, NUM_USERS, U_PAD - NUM_USERS)
    s_val = _pack(s_vals, E_S_PAD)

    ep1, sp1 = _spmm_pair(a_src, a_dst, a_val, s_src, s_dst, s_val,
                          ego0, soc0, zeros)
    ego1, soc1 = _fusion_mid(ep1, sp1, fc1_w, fc1_b, fc2_w, fc2_b, fc3_w, fc3_b)
    ep2, sp2 = _spmm_pair(a_src, a_dst, a_val, s_src, s_dst, s_val,
                          ego1, soc1, zeros)
    u_mean, i_mean = _fusion_final(ep2, sp2, fc1_w, fc1_b, fc2_w, fc2_b,
                                   fc3_w, fc3_b, ego0, ego1)
    return u_mean, i_mean
